# Initial kernel scaffold; baseline (speedup 1.0000x reference)
import jax
import jax.numpy as jnp
from jax.experimental import pallas as pl
from jax.experimental.pallas import tpu as pltpu

_B, _F, _T = 8, 128, 16384
_C = 512
_BG = 2
_NB = _B // _BG
_NT = _T // _C


def _pcen_body(sc_ref, x0_ref, d_ref, o_ref, w_ref, p_ref, carry_ref):
    t = pl.program_id(1)
    s = sc_ref[0]
    alpha = sc_ref[1]
    rabs = sc_ref[2]
    delta = sc_ref[3]
    eps = sc_ref[4]

    @pl.when(t == 0)
    def _init():
        ti = jax.lax.broadcasted_iota(jnp.float32, (_C, _C), 1)
        ji = jax.lax.broadcasted_iota(jnp.float32, (_C, _C), 0)
        d = ti - ji
        ln = jnp.log1p(-s)
        w_ref[...] = jnp.where(
            d == 0.0, s, jnp.where(d > 0.0, s * jnp.exp(d * ln), 0.0)
        )
        tt = jax.lax.broadcasted_iota(jnp.float32, (1, _C), 1)
        p_ref[...] = jnp.exp((tt + 1.0) * ln)
        carry_ref[...] = jnp.zeros_like(carry_ref)

    x0 = x0_ref[...]
    m = jax.lax.dot_general(
        x0, w_ref[...], (((1,), (0,)), ((), ())),
        preferred_element_type=jnp.float32,
        precision=jax.lax.Precision.HIGHEST,
    )
    m = m + carry_ref[...] * p_ref[...]
    carry_ref[...] = m[:, _C - 1:_C]

    madde = m + eps
    mepow = jnp.sign(madde) * jnp.exp(alpha * jnp.log(jnp.abs(madde)))
    inv = 1.0 / mepow
    dpow = jnp.exp(rabs * jnp.log(delta))

    y = d_ref[...] * inv[None, :, :] + delta
    o_ref[...] = jnp.exp(rabs * jnp.log(y)) - dpow


def kernel(data, alpha, r, delta, s, eps):
    x0 = data[0]
    scalars = jnp.stack(
        [s, alpha, jnp.abs(r), delta, eps]
    ).astype(jnp.float32)

    return pl.pallas_call(
        _pcen_body,
        out_shape=jax.ShapeDtypeStruct((_B, _F, _T), jnp.float32),
        grid=(_BG, _NT),
        in_specs=[
            pl.BlockSpec(memory_space=pltpu.SMEM),
            pl.BlockSpec((_F, _C), lambda bg, t: (0, t)),
            pl.BlockSpec((_NB, _F, _C), lambda bg, t: (bg, 0, t)),
        ],
        out_specs=pl.BlockSpec((_NB, _F, _C), lambda bg, t: (bg, 0, t)),
        scratch_shapes=[
            pltpu.VMEM((_C, _C), jnp.float32),
            pltpu.VMEM((1, _C), jnp.float32),
            pltpu.VMEM((_F, 1), jnp.float32),
        ],
        compiler_params=pltpu.CompilerParams(
            dimension_semantics=("parallel", "arbitrary"),
        ),
        name="pcen_fused",
    )(scalars, x0, data)


# speedup vs baseline: 413.3009x; 413.3009x over previous
import jax
import jax.numpy as jnp
from jax.experimental import pallas as pl
from jax.experimental.pallas import tpu as pltpu

_B, _F, _T = 8, 128, 16384
_C = 512
_BG = 2
_NB = _B // _BG
_NT = _T // _C


def _pcen_body(sc_ref, x0_ref, d_ref, o_ref, w_ref, p_ref, carry_ref):
    t = pl.program_id(1)
    s = sc_ref[0]
    alpha = sc_ref[1]
    rabs = sc_ref[2]
    delta = sc_ref[3]
    eps = sc_ref[4]

    @pl.when(t == 0)
    def _init():
        ti = jax.lax.broadcasted_iota(jnp.int32, (_C, _C), 1)
        ji = jax.lax.broadcasted_iota(jnp.int32, (_C, _C), 0)
        d = (ti - ji).astype(jnp.float32)
        ln = jnp.log1p(-s)
        w_ref[...] = jnp.where(
            d == 0.0, s, jnp.where(d > 0.0, s * jnp.exp(d * ln), 0.0)
        )
        tt = jax.lax.broadcasted_iota(jnp.int32, (1, _C), 1).astype(jnp.float32)
        p_ref[...] = jnp.exp((tt + 1.0) * ln)
        carry_ref[...] = jnp.zeros_like(carry_ref)

    x0 = x0_ref[...]
    m = jax.lax.dot_general(
        x0, w_ref[...], (((1,), (0,)), ((), ())),
        preferred_element_type=jnp.float32,
        precision=jax.lax.Precision.HIGHEST,
    )
    m = m + carry_ref[...] * p_ref[...]
    carry_ref[...] = m[:, _C - 1:_C]

    madde = m + eps
    mepow = jnp.sign(madde) * jnp.exp(alpha * jnp.log(jnp.abs(madde)))
    inv = 1.0 / mepow
    dpow = jnp.exp(rabs * jnp.log(delta))

    y = d_ref[...] * inv[None, :, :] + delta
    o_ref[...] = jnp.exp(rabs * jnp.log(y)) - dpow


def kernel(data, alpha, r, delta, s, eps):
    x0 = data[0]
    scalars = jnp.stack(
        [s, alpha, jnp.abs(r), delta, eps]
    ).astype(jnp.float32)

    return pl.pallas_call(
        _pcen_body,
        out_shape=jax.ShapeDtypeStruct((_B, _F, _T), jnp.float32),
        grid=(_BG, _NT),
        in_specs=[
            pl.BlockSpec(memory_space=pltpu.SMEM),
            pl.BlockSpec((_F, _C), lambda bg, t: (0, t)),
            pl.BlockSpec((_NB, _F, _C), lambda bg, t: (bg, 0, t)),
        ],
        out_specs=pl.BlockSpec((_NB, _F, _C), lambda bg, t: (bg, 0, t)),
        scratch_shapes=[
            pltpu.VMEM((_C, _C), jnp.float32),
            pltpu.VMEM((1, _C), jnp.float32),
            pltpu.VMEM((_F, 1), jnp.float32),
        ],
        compiler_params=pltpu.CompilerParams(
            dimension_semantics=("parallel", "arbitrary"),
        ),
        name="pcen_fused",
    )(scalars, x0, data)


# speedup vs baseline: 597.0007x; 1.4445x over previous
import jax
import jax.numpy as jnp
from jax.experimental import pallas as pl
from jax.experimental.pallas import tpu as pltpu

_B, _F, _T = 8, 128, 16384
_C = 512
_NT = _T // _C


def _pcen_body(sc_ref, x0_ref, d_ref, o_ref, w_ref, p_ref, carry_ref):
    t = pl.program_id(0)
    s = sc_ref[0]
    alpha = sc_ref[1]
    rabs = sc_ref[2]
    delta = sc_ref[3]
    eps = sc_ref[4]

    @pl.when(t == 0)
    def _init():
        ti = jax.lax.broadcasted_iota(jnp.int32, (_C, _C), 1)
        ji = jax.lax.broadcasted_iota(jnp.int32, (_C, _C), 0)
        d = (ti - ji).astype(jnp.float32)
        ln = jnp.log1p(-s)
        w_ref[...] = jnp.where(
            d == 0.0, s, jnp.where(d > 0.0, s * jnp.exp(d * ln), 0.0)
        )
        tt = jax.lax.broadcasted_iota(jnp.int32, (1, _C), 1).astype(jnp.float32)
        p_ref[...] = jnp.exp((tt + 1.0) * ln)
        carry_ref[...] = jnp.zeros_like(carry_ref)

    x0 = x0_ref[...]
    m = jax.lax.dot_general(
        x0, w_ref[...], (((1,), (0,)), ((), ())),
        preferred_element_type=jnp.float32,
    )
    m = m + carry_ref[...] * p_ref[...]
    carry_ref[...] = m[:, _C - 1:_C]

    madde = m + eps
    inv = jnp.exp(alpha * -jnp.log(madde))
    y = d_ref[...] * inv[None, :, :] + delta

    @pl.when(rabs == 0.5)
    def _sqrt_path():
        o_ref[...] = jnp.sqrt(y) - jnp.sqrt(delta)

    @pl.when(rabs != 0.5)
    def _pow_path():
        dpow = jnp.exp(rabs * jnp.log(delta))
        o_ref[...] = jnp.exp(rabs * jnp.log(y)) - dpow


def kernel(data, alpha, r, delta, s, eps):
    x0 = data[0]
    scalars = jnp.stack(
        [s, alpha, jnp.abs(r), delta, eps]
    ).astype(jnp.float32)

    return pl.pallas_call(
        _pcen_body,
        out_shape=jax.ShapeDtypeStruct((_B, _F, _T), jnp.float32),
        grid=(_NT,),
        in_specs=[
            pl.BlockSpec(memory_space=pltpu.SMEM),
            pl.BlockSpec((_F, _C), lambda t: (0, t)),
            pl.BlockSpec((_B, _F, _C), lambda t: (0, 0, t)),
        ],
        out_specs=pl.BlockSpec((_B, _F, _C), lambda t: (0, 0, t)),
        scratch_shapes=[
            pltpu.VMEM((_C, _C), jnp.float32),
            pltpu.VMEM((1, _C), jnp.float32),
            pltpu.VMEM((_F, 1), jnp.float32),
        ],
        compiler_params=pltpu.CompilerParams(
            dimension_semantics=("arbitrary",),
        ),
        name="pcen_fused",
    )(scalars, x0, data)


# speedup vs baseline: 703.5601x; 1.1785x over previous
import jax
import jax.numpy as jnp
from jax.experimental import pallas as pl
from jax.experimental.pallas import tpu as pltpu

_B, _F, _T = 8, 128, 16384
_C = 512
_NT = _T // _C


def _pcen_body(sc_ref, d_ref, o_ref, w_ref, p_ref, carry_ref):
    t = pl.program_id(0)
    s = sc_ref[0]
    alpha = sc_ref[1]
    rabs = sc_ref[2]
    delta = sc_ref[3]
    eps = sc_ref[4]

    @pl.when(t == 0)
    def _init():
        ti = jax.lax.broadcasted_iota(jnp.int32, (_C, _C), 1)
        ji = jax.lax.broadcasted_iota(jnp.int32, (_C, _C), 0)
        d = (ti - ji).astype(jnp.float32)
        ln = jnp.log1p(-s)
        w_ref[...] = jnp.where(
            d == 0.0, s, jnp.where(d > 0.0, s * jnp.exp(d * ln), 0.0)
        )
        tt = jax.lax.broadcasted_iota(jnp.int32, (1, _C), 1).astype(jnp.float32)
        p_ref[...] = jnp.exp((tt + 1.0) * ln)
        carry_ref[...] = jnp.zeros_like(carry_ref)

    x0 = d_ref[0]
    m = jax.lax.dot_general(
        x0, w_ref[...], (((1,), (0,)), ((), ())),
        preferred_element_type=jnp.float32,
    )
    m = m + carry_ref[...] * p_ref[...]
    carry_ref[...] = m[:, _C - 1:_C]

    madde = m + eps
    inv = jnp.exp(alpha * -jnp.log(madde))
    y = d_ref[...] * inv[None, :, :] + delta

    @pl.when(rabs == 0.5)
    def _sqrt_path():
        o_ref[...] = jnp.sqrt(y) - jnp.sqrt(delta)

    @pl.when(rabs != 0.5)
    def _pow_path():
        dpow = jnp.exp(rabs * jnp.log(delta))
        o_ref[...] = jnp.exp(rabs * jnp.log(y)) - dpow


def kernel(data, alpha, r, delta, s, eps):
    scalars = jnp.stack(
        [s, alpha, jnp.abs(r), delta, eps]
    ).astype(jnp.float32)

    return pl.pallas_call(
        _pcen_body,
        out_shape=jax.ShapeDtypeStruct((_B, _F, _T), jnp.float32),
        grid=(_NT,),
        in_specs=[
            pl.BlockSpec(memory_space=pltpu.SMEM),
            pl.BlockSpec((_B, _F, _C), lambda t: (0, 0, t)),
        ],
        out_specs=pl.BlockSpec((_B, _F, _C), lambda t: (0, 0, t)),
        scratch_shapes=[
            pltpu.VMEM((_C, _C), jnp.float32),
            pltpu.VMEM((1, _C), jnp.float32),
            pltpu.VMEM((_F, 1), jnp.float32),
        ],
        compiler_params=pltpu.CompilerParams(
            dimension_semantics=("arbitrary",),
        ),
        name="pcen_fused",
    )(scalars, data)


# speedup vs baseline: 792.8613x; 1.1269x over previous
import jax
import jax.numpy as jnp
from jax.experimental import pallas as pl
from jax.experimental.pallas import tpu as pltpu

_B, _F, _T = 8, 128, 16384
_C = 1024
_NT = _T // _C


def _pcen_body(sc_ref, d_ref, o_ref, w_ref, p_ref, carry_ref):
    t = pl.program_id(0)
    s = sc_ref[0]
    alpha = sc_ref[1]
    rabs = sc_ref[2]
    delta = sc_ref[3]
    eps = sc_ref[4]

    @pl.when(t == 0)
    def _init():
        ti = jax.lax.broadcasted_iota(jnp.int32, (_C, _C), 1)
        ji = jax.lax.broadcasted_iota(jnp.int32, (_C, _C), 0)
        d = (ti - ji).astype(jnp.float32)
        ln = jnp.log1p(-s)
        w_ref[...] = jnp.where(
            d == 0.0, s, jnp.where(d > 0.0, s * jnp.exp(d * ln), 0.0)
        )
        tt = jax.lax.broadcasted_iota(jnp.int32, (1, _C), 1).astype(jnp.float32)
        p_ref[...] = jnp.exp((tt + 1.0) * ln)
        carry_ref[...] = jnp.zeros_like(carry_ref)

    x0 = d_ref[0]
    m = jax.lax.dot_general(
        x0, w_ref[...], (((1,), (0,)), ((), ())),
        preferred_element_type=jnp.float32,
    )
    m = m + carry_ref[...] * p_ref[...]
    carry_ref[...] = m[:, _C - 1:_C]

    madde = m + eps
    inv = jnp.exp(alpha * -jnp.log(madde))
    y = d_ref[...] * inv[None, :, :] + delta

    @pl.when(rabs == 0.5)
    def _sqrt_path():
        o_ref[...] = jnp.sqrt(y) - jnp.sqrt(delta)

    @pl.when(rabs != 0.5)
    def _pow_path():
        dpow = jnp.exp(rabs * jnp.log(delta))
        o_ref[...] = jnp.exp(rabs * jnp.log(y)) - dpow


def kernel(data, alpha, r, delta, s, eps):
    scalars = jnp.stack(
        [s, alpha, jnp.abs(r), delta, eps]
    ).astype(jnp.float32)

    return pl.pallas_call(
        _pcen_body,
        out_shape=jax.ShapeDtypeStruct((_B, _F, _T), jnp.float32),
        grid=(_NT,),
        in_specs=[
            pl.BlockSpec(memory_space=pltpu.SMEM),
            pl.BlockSpec((_B, _F, _C), lambda t: (0, 0, t)),
        ],
        out_specs=pl.BlockSpec((_B, _F, _C), lambda t: (0, 0, t)),
        scratch_shapes=[
            pltpu.VMEM((_C, _C), jnp.float32),
            pltpu.VMEM((1, _C), jnp.float32),
            pltpu.VMEM((_F, 1), jnp.float32),
        ],
        compiler_params=pltpu.CompilerParams(
            dimension_semantics=("arbitrary",),
        ),
        name="pcen_fused",
    )(scalars, data)
